# baseline (speedup 1.0000x reference)
import jax
import jax.numpy as jnp
from jax.experimental import pallas as pl
from jax.experimental.pallas import tpu as pltpu

_SEQ = 20
_N = 256
_RNN = 128
_G = 4
_G2 = _G * _G
_EMB = 64
_INP = 2
_OUT = 5
_PXPAD = 8


def _perm_apply(P, X, transpose):
    f32 = jnp.float32
    p1 = X.astype(jnp.bfloat16).astype(f32)
    r1 = X - p1
    p2 = r1.astype(jnp.bfloat16).astype(f32)
    p3 = r1 - p2
    if transpose:
        dn = (((0,), (0,)), ((), ()))
    else:
        dn = (((1,), (0,)), ((), ()))
    acc = jax.lax.dot_general(P, p1, dn, preferred_element_type=f32)
    acc = acc + jax.lax.dot_general(P, p2, dn, preferred_element_type=f32)
    acc = acc + jax.lax.dot_general(P, p3, dn, preferred_element_type=f32)
    return acc


def _social_lstm_kernel(gt_ref, pedxy_ref, ids_ref, h0_ref, c0_ref,
                        win_ref, bin_ref, wt_ref, bt_ref,
                        wih_ref, bih_ref, whh_ref, bhh_ref, wo_ref, bo_ref,
                        out_ref, hf_ref, cf_ref,
                        social_ref, concat_ref):
    t = pl.program_id(0)
    f32 = jnp.float32

    @pl.when(t == 0)
    def _init():
        hf_ref[...] = h0_ref[...]
        cf_ref[...] = c0_ref[...]

    ids = ids_ref[0]
    iota = jax.lax.broadcasted_iota(jnp.int32, (_N, _N), 0)
    P = (iota == ids).astype(f32)

    h_cur = _perm_apply(P, hf_ref[...], True)
    c_cur = _perm_apply(P, cf_ref[...], True)
    px = _perm_apply(P, pedxy_ref[0], True)

    inp_emb = jnp.maximum(
        jnp.dot(px, win_ref[...], preferred_element_type=f32) + bin_ref[...],
        0.0)

    for g in range(_G2):
        social_ref[:, g * _RNN:(g + 1) * _RNN] = jnp.dot(
            gt_ref[0, g], h_cur, preferred_element_type=f32)
    ten_emb = jnp.maximum(
        jnp.dot(social_ref[...], wt_ref[...], preferred_element_type=f32)
        + bt_ref[...], 0.0)

    concat_ref[:, :_EMB] = inp_emb
    concat_ref[:, _EMB:] = ten_emb
    gates = ((jnp.dot(concat_ref[...], wih_ref[...], preferred_element_type=f32)
              + bih_ref[...])
             + jnp.dot(h_cur, whh_ref[...], preferred_element_type=f32)
             ) + bhh_ref[...]
    i = jax.nn.sigmoid(gates[:, 0 * _RNN:1 * _RNN])
    f = jax.nn.sigmoid(gates[:, 1 * _RNN:2 * _RNN])
    g_ = jnp.tanh(gates[:, 2 * _RNN:3 * _RNN])
    o = jax.nn.sigmoid(gates[:, 3 * _RNN:4 * _RNN])
    c_new = f * c_cur + i * g_
    h_new = o * jnp.tanh(c_new)
    out_t = jnp.dot(h_new, wo_ref[...], preferred_element_type=f32) + bo_ref[...]

    out_ref[0] = _perm_apply(P, out_t, False)
    hf_ref[...] = _perm_apply(P, h_new, False)
    cf_ref[...] = _perm_apply(P, c_new, False)


def kernel(pedxy, hidden_states, cell_states, outputs, grids, node_ids,
           W_in, b_in, W_t, b_t, W_ih, b_ih, W_hh, b_hh, W_out, b_out):
    del outputs

    grids_t = jnp.concatenate(
        [jnp.transpose(grids[:_SEQ // 2], (0, 3, 1, 2)),
         jnp.transpose(grids[_SEQ // 2:], (0, 3, 1, 2))], axis=0)
    pedxy8 = jnp.pad(pedxy, ((0, 0), (0, 0), (0, _PXPAD - _INP)))
    win8 = jnp.pad(W_in, ((0, _PXPAD - _INP), (0, 0)))
    ids3 = node_ids.reshape(_SEQ, 1, _N)

    const = lambda *shape: pl.BlockSpec(shape, lambda t: (0,) * len(shape))
    out_shapes = (
        jax.ShapeDtypeStruct((_SEQ, _N, _OUT), jnp.float32),
        jax.ShapeDtypeStruct((_N, _RNN), jnp.float32),
        jax.ShapeDtypeStruct((_N, _RNN), jnp.float32),
    )
    outs = pl.pallas_call(
        _social_lstm_kernel,
        grid=(_SEQ,),
        in_specs=[
            pl.BlockSpec((1, _G2, _N, _N), lambda t: (t, 0, 0, 0)),
            pl.BlockSpec((1, _N, _PXPAD), lambda t: (t, 0, 0)),
            pl.BlockSpec((1, 1, _N), lambda t: (t, 0, 0)),
            const(_N, _RNN),
            const(_N, _RNN),
            const(_PXPAD, _EMB),
            const(1, _EMB),
            const(_G2 * _RNN, _EMB),
            const(1, _EMB),
            const(2 * _EMB, 4 * _RNN),
            const(1, 4 * _RNN),
            const(_RNN, 4 * _RNN),
            const(1, 4 * _RNN),
            const(_RNN, _OUT),
            const(1, _OUT),
        ],
        out_specs=[
            pl.BlockSpec((1, _N, _OUT), lambda t: (t, 0, 0)),
            const(_N, _RNN),
            const(_N, _RNN),
        ],
        out_shape=out_shapes,
        scratch_shapes=[
            pltpu.VMEM((_N, _G2 * _RNN), jnp.float32),
            pltpu.VMEM((_N, 2 * _EMB), jnp.float32),
        ],
        compiler_params=pltpu.CompilerParams(
            dimension_semantics=("arbitrary",),
        ),
    )(grids_t, pedxy8, ids3, hidden_states, cell_states,
      win8, b_in.reshape(1, _EMB), W_t, b_t.reshape(1, _EMB),
      W_ih.T, b_ih.reshape(1, 4 * _RNN), W_hh.T, b_hh.reshape(1, 4 * _RNN),
      W_out.T, b_out.reshape(1, _OUT))
    return outs[0], outs[1], outs[2]


# speedup vs baseline: 1.7630x; 1.7630x over previous
import jax
import jax.numpy as jnp
from jax.experimental import pallas as pl
from jax.experimental.pallas import tpu as pltpu

_SEQ = 20
_N = 256
_RNN = 128
_G = 4
_G2 = _G * _G
_EMB = 64
_INP = 2
_OUT = 5
_PXPAD = 8


def _perm_apply(P16, X, transpose):
    f32 = jnp.float32
    b16 = jnp.bfloat16
    p1 = X.astype(b16)
    r1 = X - p1.astype(f32)
    p2 = r1.astype(b16)
    p3 = (r1 - p2.astype(f32)).astype(b16)
    if transpose:
        dn = (((0,), (0,)), ((), ()))
    else:
        dn = (((1,), (0,)), ((), ()))
    acc = jax.lax.dot_general(P16, p1, dn, preferred_element_type=f32)
    acc = acc + jax.lax.dot_general(P16, p2, dn, preferred_element_type=f32)
    acc = acc + jax.lax.dot_general(P16, p3, dn, preferred_element_type=f32)
    return acc


def _social_lstm_kernel(gt_ref, pedxy_ref, ids_ref, h0_ref, c0_ref,
                        win_ref, bin_ref, wt_ref, bt_ref,
                        wih_ref, bih_ref, whh_ref, bhh_ref, wo_ref, bo_ref,
                        out_ref, hf_ref, cf_ref,
                        social_ref, concat_ref):
    t = pl.program_id(0)
    f32 = jnp.float32

    @pl.when(t == 0)
    def _init():
        hf_ref[...] = h0_ref[...]
        cf_ref[...] = c0_ref[...]

    ids = ids_ref[0]
    iota = jax.lax.broadcasted_iota(jnp.int32, (_N, _N), 0)
    P = (iota == ids).astype(jnp.bfloat16)

    hc = _perm_apply(P, jnp.concatenate([hf_ref[...], cf_ref[...]], axis=1),
                     True)
    h_cur = hc[:, :_RNN]
    c_cur = hc[:, _RNN:]
    px = _perm_apply(P, pedxy_ref[0], True)

    inp_emb = jnp.maximum(
        jnp.dot(px, win_ref[...], preferred_element_type=f32) + bin_ref[...],
        0.0)

    for g in range(_G2):
        social_ref[:, g * _RNN:(g + 1) * _RNN] = jnp.dot(
            gt_ref[0, g], h_cur, preferred_element_type=f32)
    ten_emb = jnp.maximum(
        jnp.dot(social_ref[...], wt_ref[...], preferred_element_type=f32)
        + bt_ref[...], 0.0)

    concat_ref[:, :_EMB] = inp_emb
    concat_ref[:, _EMB:] = ten_emb
    gates = ((jnp.dot(concat_ref[...], wih_ref[...], preferred_element_type=f32)
              + bih_ref[...])
             + jnp.dot(h_cur, whh_ref[...], preferred_element_type=f32)
             ) + bhh_ref[...]
    i = jax.nn.sigmoid(gates[:, 0 * _RNN:1 * _RNN])
    f = jax.nn.sigmoid(gates[:, 1 * _RNN:2 * _RNN])
    g_ = jnp.tanh(gates[:, 2 * _RNN:3 * _RNN])
    o = jax.nn.sigmoid(gates[:, 3 * _RNN:4 * _RNN])
    c_new = f * c_cur + i * g_
    h_new = o * jnp.tanh(c_new)
    out_t = jnp.dot(h_new, wo_ref[...], preferred_element_type=f32) + bo_ref[...]

    out_ref[0] = _perm_apply(P, out_t, False)
    hc_new = _perm_apply(P, jnp.concatenate([h_new, c_new], axis=1), False)
    hf_ref[...] = hc_new[:, :_RNN]
    cf_ref[...] = hc_new[:, _RNN:]


def kernel(pedxy, hidden_states, cell_states, outputs, grids, node_ids,
           W_in, b_in, W_t, b_t, W_ih, b_ih, W_hh, b_hh, W_out, b_out):
    del outputs

    grids_t = jnp.transpose(grids, (0, 3, 1, 2))
    pedxy8 = jnp.pad(pedxy, ((0, 0), (0, 0), (0, _PXPAD - _INP)))
    win8 = jnp.pad(W_in, ((0, _PXPAD - _INP), (0, 0)))
    ids3 = node_ids.reshape(_SEQ, 1, _N)

    const = lambda *shape: pl.BlockSpec(shape, lambda t: (0,) * len(shape))
    out_shapes = (
        jax.ShapeDtypeStruct((_SEQ, _N, _OUT), jnp.float32),
        jax.ShapeDtypeStruct((_N, _RNN), jnp.float32),
        jax.ShapeDtypeStruct((_N, _RNN), jnp.float32),
    )
    outs = pl.pallas_call(
        _social_lstm_kernel,
        grid=(_SEQ,),
        in_specs=[
            pl.BlockSpec((1, _G2, _N, _N), lambda t: (t, 0, 0, 0)),
            pl.BlockSpec((1, _N, _PXPAD), lambda t: (t, 0, 0)),
            pl.BlockSpec((1, 1, _N), lambda t: (t, 0, 0)),
            const(_N, _RNN),
            const(_N, _RNN),
            const(_PXPAD, _EMB),
            const(1, _EMB),
            const(_G2 * _RNN, _EMB),
            const(1, _EMB),
            const(2 * _EMB, 4 * _RNN),
            const(1, 4 * _RNN),
            const(_RNN, 4 * _RNN),
            const(1, 4 * _RNN),
            const(_RNN, _OUT),
            const(1, _OUT),
        ],
        out_specs=[
            pl.BlockSpec((1, _N, _OUT), lambda t: (t, 0, 0)),
            const(_N, _RNN),
            const(_N, _RNN),
        ],
        out_shape=out_shapes,
        scratch_shapes=[
            pltpu.VMEM((_N, _G2 * _RNN), jnp.float32),
            pltpu.VMEM((_N, 2 * _EMB), jnp.float32),
        ],
        compiler_params=pltpu.CompilerParams(
            dimension_semantics=("arbitrary",),
        ),
    )(grids_t, pedxy8, ids3, hidden_states, cell_states,
      win8, b_in.reshape(1, _EMB), W_t, b_t.reshape(1, _EMB),
      W_ih.T, b_ih.reshape(1, 4 * _RNN), W_hh.T, b_hh.reshape(1, 4 * _RNN),
      W_out.T, b_out.reshape(1, _OUT))
    return outs[0], outs[1], outs[2]
